# Initial kernel scaffold; baseline (speedup 1.0000x reference)
import jax
import jax.numpy as jnp
from jax.experimental import pallas as pl


def kernel(x, edge_index, batch, W1, b1, W2, b2, W3, b3, fc1_W, fc1_b, fc2_W, fc2_b):
    raise NotImplementedError("write your pallas kernel here")


# speedup vs baseline: 17.9409x; 17.9409x over previous
import functools

import jax
import jax.numpy as jnp
from jax import lax
from jax.experimental import pallas as pl
from jax.experimental.pallas import tpu as pltpu
from jax.experimental.pallas import tpu_sc as plsc

_N = 10000
_E = 320000
_G = 64
_DH = 128
_CHUNK = 128


def _mesh():
    return plsc.VectorSubcoreMesh(core_axis_name="c", subcore_axis_name="s")


def _row_split(ns):
    base = (_N // 8 // ns) * 8
    tail = _N - base * ns
    return base, tail


def _zero_acc(zeros_hbm, acc_sh, s, ns):
    base, tail = _row_split(ns)
    pltpu.sync_copy(zeros_hbm.at[pl.ds(0, base)], acc_sh.at[pl.ds(s * base, base)])
    if tail:
        @pl.when(s == ns - 1)
        def _():
            pltpu.sync_copy(
                zeros_hbm.at[pl.ds(0, tail)], acc_sh.at[pl.ds(ns * base, tail)]
            )


def _copy_out(acc_sh, out_hbm, c, s, ns):
    base, tail = _row_split(ns)
    pltpu.sync_copy(
        acc_sh.at[pl.ds(s * base, base)], out_hbm.at[c, pl.ds(s * base, base)]
    )
    if tail:
        @pl.when(s == ns - 1)
        def _():
            pltpu.sync_copy(
                acc_sh.at[pl.ds(ns * base, tail)],
                out_hbm.at[c, pl.ds(ns * base, tail)],
            )


def _sc_degree(nc, ns, ki):
    @functools.partial(
        pl.kernel,
        out_type=jax.ShapeDtypeStruct((nc, _N, _DH), jnp.float32),
        mesh=_mesh(),
        scratch_types=[
            pltpu.VMEM((ki, _CHUNK), jnp.int32),
            pltpu.VMEM((_CHUNK, _DH), jnp.float32),
            pltpu.VMEM_SHARED((_N + _CHUNK, _DH), jnp.float32),
        ],
    )
    def deg_kernel(dstb_hbm, ones_hbm, zeros_hbm, out_hbm, idx_v, ones_v, acc_sh):
        c = lax.axis_index("c")
        s = lax.axis_index("s")
        wid = s * nc + c
        _zero_acc(zeros_hbm, acc_sh, s, ns)
        pltpu.sync_copy(dstb_hbm.at[wid], idx_v)
        pltpu.sync_copy(ones_hbm, ones_v)
        plsc.subcore_barrier()

        def body(j, carry):
            pltpu.sync_copy(ones_v, acc_sh.at[idx_v.at[j]], add=True)
            return carry

        lax.fori_loop(0, ki, body, 0)
        plsc.subcore_barrier()
        _copy_out(acc_sh, out_hbm, c, s, ns)

    return deg_kernel


def _sc_edge(nc, ns, ki):
    @functools.partial(
        pl.kernel,
        out_type=jax.ShapeDtypeStruct((nc, _N, _DH), jnp.float32),
        mesh=_mesh(),
        scratch_types=[
            pltpu.VMEM((ki, _CHUNK), jnp.int32),
            pltpu.VMEM((ki, _CHUNK), jnp.int32),
            pltpu.VMEM((_CHUNK, _DH), jnp.float32),
            pltpu.VMEM_SHARED((_N + _CHUNK, _DH), jnp.float32),
        ],
    )
    def edge_kernel(hp_hbm, srcb_hbm, dstb_hbm, zeros_hbm, out_hbm,
                    sidx_v, didx_v, rows_v, acc_sh):
        c = lax.axis_index("c")
        s = lax.axis_index("s")
        wid = s * nc + c
        _zero_acc(zeros_hbm, acc_sh, s, ns)
        pltpu.sync_copy(srcb_hbm.at[wid], sidx_v)
        pltpu.sync_copy(dstb_hbm.at[wid], didx_v)
        plsc.subcore_barrier()

        def body(j, carry):
            pltpu.sync_copy(hp_hbm.at[sidx_v.at[j]], rows_v)
            pltpu.sync_copy(rows_v, acc_sh.at[didx_v.at[j]], add=True)
            return carry

        lax.fori_loop(0, ki, body, 0)
        plsc.subcore_barrier()
        _copy_out(acc_sh, out_hbm, c, s, ns)

    return edge_kernel


def _dinv(degp_ref, nc):
    deg = degp_ref[0, :, 0:1]
    for i in range(1, nc):
        deg = deg + degp_ref[i, :, 0:1]
    return lax.rsqrt(deg + 1.0)


def _tc_first(x, W1, degp, nc):
    def body(x_ref, w_ref, degp_ref, out_ref):
        dinv = _dinv(degp_ref, nc)
        h = jnp.dot(x_ref[...], w_ref[...], preferred_element_type=jnp.float32)
        out_ref[...] = h * dinv

    return pl.pallas_call(
        body, out_shape=jax.ShapeDtypeStruct((_N, _DH), jnp.float32)
    )(x, W1, degp)


def _tc_mid(accp, hp, degp, b, W, nc):
    def body(acc_ref, hp_ref, degp_ref, b_ref, w_ref, out_ref):
        dinv = _dinv(degp_ref, nc)
        pre = hp_ref[...]
        for i in range(nc):
            pre = pre + acc_ref[i]
        a = jnp.maximum(dinv * pre + b_ref[...], 0.0)
        h = jnp.dot(a, w_ref[...], preferred_element_type=jnp.float32)
        out_ref[...] = h * dinv

    return pl.pallas_call(
        body, out_shape=jax.ShapeDtypeStruct((_N, _DH), jnp.float32)
    )(accp, hp, degp, b, W)


def _tc_final(accp, hp, degp, b3, batch2, fc1_W, fc1_b, fc2_W, fc2_b, nc):
    def body(acc_ref, hp_ref, degp_ref, b_ref, batch_ref, w1_ref, b1_ref,
             w2_ref, b2_ref, out_ref):
        dinv = _dinv(degp_ref, nc)
        pre = hp_ref[...]
        for i in range(nc):
            pre = pre + acc_ref[i]
        h = jnp.maximum(dinv * pre + b_ref[...], 0.0)
        seg = jax.lax.broadcasted_iota(jnp.int32, (_G, _N), 0) == batch_ref[...]
        pooled = jnp.dot(
            seg.astype(jnp.float32), h, preferred_element_type=jnp.float32
        )
        g = jnp.maximum(
            jnp.dot(pooled, w1_ref[...], preferred_element_type=jnp.float32)
            + b1_ref[...],
            0.0,
        )
        out_ref[...] = (
            jnp.dot(g, w2_ref[...], preferred_element_type=jnp.float32)
            + b2_ref[...]
        )

    return pl.pallas_call(
        body, out_shape=jax.ShapeDtypeStruct((_G, fc2_W.shape[1]), jnp.float32)
    )(accp, hp, degp, b3, batch2, fc1_W, fc1_b, fc2_W, fc2_b)


def kernel(x, edge_index, batch, W1, b1, W2, b2, W3, b3, fc1_W, fc1_b, fc2_W, fc2_b):
    info = plsc.get_sparse_core_info()
    nc, ns = info.num_cores, info.num_subcores
    nw = nc * ns
    ki = -(-_E // (nw * _CHUNK))
    e_pad = nw * ki * _CHUNK
    pad = e_pad - _E

    src = edge_index[0].astype(jnp.int32)
    dst = edge_index[1].astype(jnp.int32)
    spread = jnp.arange(pad, dtype=jnp.int32)
    srcb = jnp.concatenate([src, spread * 37 % _N]).reshape(nw, ki, _CHUNK)
    dstb = jnp.concatenate([dst, _N + spread % _CHUNK]).reshape(nw, ki, _CHUNK)

    base, _ = _row_split(ns)
    onesD = jnp.ones((_CHUNK, _DH), jnp.float32)
    zerosD = jnp.zeros((base, _DH), jnp.float32)

    degp = _sc_degree(nc, ns, ki)(dstb, onesD, zerosD)

    b1r, b2r, b3r = (b.reshape(1, -1) for b in (b1, b2, b3))
    fc1_br = fc1_b.reshape(1, -1)
    fc2_br = fc2_b.reshape(1, -1)
    batch2 = batch.astype(jnp.int32).reshape(1, _N)

    edge = _sc_edge(nc, ns, ki)

    hp1 = _tc_first(x, W1, degp, nc)
    acc1 = edge(hp1, srcb, dstb, zerosD)
    hp2 = _tc_mid(acc1, hp1, degp, b1r, W2, nc)
    acc2 = edge(hp2, srcb, dstb, zerosD)
    hp3 = _tc_mid(acc2, hp2, degp, b2r, W3, nc)
    acc3 = edge(hp3, srcb, dstb, zerosD)
    out = _tc_final(acc3, hp3, degp, b3r, batch2, fc1_W, fc1_br, fc2_W, fc2_br, nc)
    return out


# speedup vs baseline: 20.1825x; 1.1249x over previous
import functools

import jax
import jax.numpy as jnp
from jax import lax
from jax.experimental import pallas as pl
from jax.experimental.pallas import tpu as pltpu
from jax.experimental.pallas import tpu_sc as plsc

_N = 10000
_E = 320000
_G = 64
_DH = 128
_CHUNK = 128


def _mesh():
    return plsc.VectorSubcoreMesh(core_axis_name="c", subcore_axis_name="s")


def _row_split(ns):
    base = (_N // 8 // ns) * 8
    tail = _N - base * ns
    return base, tail


def _zero_acc(zeros_hbm, acc_sh, s, ns):
    base, tail = _row_split(ns)
    pltpu.sync_copy(zeros_hbm.at[pl.ds(0, base)], acc_sh.at[pl.ds(s * base, base)])
    if tail:
        @pl.when(s == ns - 1)
        def _():
            pltpu.sync_copy(
                zeros_hbm.at[pl.ds(0, tail)], acc_sh.at[pl.ds(ns * base, tail)]
            )


def _copy_out(acc_sh, out_hbm, c, s, ns):
    base, tail = _row_split(ns)
    pltpu.sync_copy(
        acc_sh.at[pl.ds(s * base, base)], out_hbm.at[c, pl.ds(s * base, base)]
    )
    if tail:
        @pl.when(s == ns - 1)
        def _():
            pltpu.sync_copy(
                acc_sh.at[pl.ds(ns * base, tail)],
                out_hbm.at[c, pl.ds(ns * base, tail)],
            )


def _sc_degree(nc, ns, ki):
    @functools.partial(
        pl.kernel,
        out_type=jax.ShapeDtypeStruct((nc, _N, _DH), jnp.float32),
        mesh=_mesh(),
        scratch_types=[
            pltpu.VMEM((ki, _CHUNK), jnp.int32),
            pltpu.VMEM((_CHUNK, _DH), jnp.float32),
            pltpu.VMEM_SHARED((_N + _CHUNK, _DH), jnp.float32),
            pltpu.SemaphoreType.DMA,
        ],
    )
    def deg_kernel(dstb_hbm, ones_hbm, zeros_hbm, out_hbm, idx_v, ones_v,
                   acc_sh, sem):
        c = lax.axis_index("c")
        s = lax.axis_index("s")
        wid = s * nc + c
        _zero_acc(zeros_hbm, acc_sh, s, ns)
        pltpu.sync_copy(dstb_hbm.at[wid], idx_v)
        pltpu.sync_copy(ones_hbm, ones_v)
        plsc.subcore_barrier()

        batch = 8
        assert ki % batch == 0

        def body(w, carry):
            ds = [
                pltpu.async_copy(
                    ones_v, acc_sh.at[idx_v.at[w * batch + u]], sem, add=True
                )
                for u in range(batch)
            ]
            for d in ds:
                d.wait()
            return carry

        lax.fori_loop(0, ki // batch, body, 0)
        plsc.subcore_barrier()
        _copy_out(acc_sh, out_hbm, c, s, ns)

    return deg_kernel


def _sc_edge(nc, ns, ki):
    assert ki % 4 == 0
    hki = ki // 2

    @functools.partial(
        pl.kernel,
        out_type=jax.ShapeDtypeStruct((nc, _N, _DH), jnp.float32),
        mesh=_mesh(),
        scratch_types=[
            pltpu.VMEM((hki, _CHUNK), jnp.int32),
            pltpu.VMEM((hki, _CHUNK), jnp.int32),
            pltpu.VMEM((_CHUNK, _DH), jnp.float32),
            pltpu.VMEM((_CHUNK, _DH), jnp.float32),
            pltpu.VMEM_SHARED((_N + _CHUNK, _DH), jnp.float32),
            pltpu.SemaphoreType.DMA,
            pltpu.SemaphoreType.DMA,
            pltpu.SemaphoreType.DMA,
            pltpu.SemaphoreType.DMA,
        ],
    )
    def edge_kernel(hp_hbm, srcb_hbm, dstb_hbm, zeros_hbm, out_hbm,
                    sidx_v, didx_v, rows0, rows1, acc_sh,
                    sem0, sem1, sem2, sem3):
        c = lax.axis_index("c")
        s = lax.axis_index("s")
        wid = s * nc + c
        _zero_acc(zeros_hbm, acc_sh, s, ns)
        plsc.subcore_barrier()

        for half in range(2):
            pltpu.sync_copy(srcb_hbm.at[wid, pl.ds(half * hki, hki)], sidx_v)
            pltpu.sync_copy(dstb_hbm.at[wid, pl.ds(half * hki, hki)], didx_v)

            def body(p, carry):
                j0 = 2 * p
                j1 = j0 + 1
                g0 = pltpu.async_copy(hp_hbm.at[sidx_v.at[j0]], rows0, sem0)
                g1 = pltpu.async_copy(hp_hbm.at[sidx_v.at[j1]], rows1, sem1)
                g0.wait()
                s0 = pltpu.async_copy(
                    rows0, acc_sh.at[didx_v.at[j0]], sem2, add=True
                )
                g1.wait()
                s1 = pltpu.async_copy(
                    rows1, acc_sh.at[didx_v.at[j1]], sem3, add=True
                )
                s0.wait()
                s1.wait()
                return carry

            lax.fori_loop(0, hki // 2, body, 0)

        plsc.subcore_barrier()
        _copy_out(acc_sh, out_hbm, c, s, ns)

    return edge_kernel


def _dinv(degp_ref, nc):
    deg = degp_ref[0, :, 0:1]
    for i in range(1, nc):
        deg = deg + degp_ref[i, :, 0:1]
    return lax.rsqrt(deg + 1.0)


def _tc_first(x, W1, degp, nc):
    def body(x_ref, w_ref, degp_ref, out_ref):
        dinv = _dinv(degp_ref, nc)
        h = jnp.dot(x_ref[...], w_ref[...], preferred_element_type=jnp.float32)
        out_ref[...] = h * dinv

    return pl.pallas_call(
        body, out_shape=jax.ShapeDtypeStruct((_N, _DH), jnp.float32)
    )(x, W1, degp)


def _tc_mid(accp, hp, degp, b, W, nc):
    def body(acc_ref, hp_ref, degp_ref, b_ref, w_ref, out_ref):
        dinv = _dinv(degp_ref, nc)
        pre = hp_ref[...]
        for i in range(nc):
            pre = pre + acc_ref[i]
        a = jnp.maximum(dinv * pre + b_ref[...], 0.0)
        h = jnp.dot(a, w_ref[...], preferred_element_type=jnp.float32)
        out_ref[...] = h * dinv

    return pl.pallas_call(
        body, out_shape=jax.ShapeDtypeStruct((_N, _DH), jnp.float32)
    )(accp, hp, degp, b, W)


def _tc_final(accp, hp, degp, b3, batch2, fc1_W, fc1_b, fc2_W, fc2_b, nc):
    def body(acc_ref, hp_ref, degp_ref, b_ref, batch_ref, w1_ref, b1_ref,
             w2_ref, b2_ref, out_ref):
        dinv = _dinv(degp_ref, nc)
        pre = hp_ref[...]
        for i in range(nc):
            pre = pre + acc_ref[i]
        h = jnp.maximum(dinv * pre + b_ref[...], 0.0)
        seg = jax.lax.broadcasted_iota(jnp.int32, (_G, _N), 0) == batch_ref[...]
        pooled = jnp.dot(
            seg.astype(jnp.float32), h, preferred_element_type=jnp.float32
        )
        g = jnp.maximum(
            jnp.dot(pooled, w1_ref[...], preferred_element_type=jnp.float32)
            + b1_ref[...],
            0.0,
        )
        out_ref[...] = (
            jnp.dot(g, w2_ref[...], preferred_element_type=jnp.float32)
            + b2_ref[...]
        )

    return pl.pallas_call(
        body, out_shape=jax.ShapeDtypeStruct((_G, fc2_W.shape[1]), jnp.float32)
    )(accp, hp, degp, b3, batch2, fc1_W, fc1_b, fc2_W, fc2_b)


def kernel(x, edge_index, batch, W1, b1, W2, b2, W3, b3, fc1_W, fc1_b, fc2_W, fc2_b):
    info = plsc.get_sparse_core_info()
    nc, ns = info.num_cores, info.num_subcores
    nw = nc * ns
    ki = -(-_E // (nw * _CHUNK))
    ki = -(-ki // 4) * 4
    e_pad = nw * ki * _CHUNK
    pad = e_pad - _E

    src = edge_index[0].astype(jnp.int32)
    dst = edge_index[1].astype(jnp.int32)
    spread = jnp.arange(pad, dtype=jnp.int32)
    srcb = jnp.concatenate([src, spread * 37 % _N]).reshape(nw, ki, _CHUNK)
    dstb = jnp.concatenate([dst, _N + spread % _CHUNK]).reshape(nw, ki, _CHUNK)

    base, _ = _row_split(ns)
    onesD = jnp.ones((_CHUNK, _DH), jnp.float32)
    zerosD = jnp.zeros((base, _DH), jnp.float32)

    degp = _sc_degree(nc, ns, ki)(dstb, onesD, zerosD)

    b1r, b2r, b3r = (b.reshape(1, -1) for b in (b1, b2, b3))
    fc1_br = fc1_b.reshape(1, -1)
    fc2_br = fc2_b.reshape(1, -1)
    batch2 = batch.astype(jnp.int32).reshape(1, _N)

    edge = _sc_edge(nc, ns, ki)

    hp1 = _tc_first(x, W1, degp, nc)
    acc1 = edge(hp1, srcb, dstb, zerosD)
    hp2 = _tc_mid(acc1, hp1, degp, b1r, W2, nc)
    acc2 = edge(hp2, srcb, dstb, zerosD)
    hp3 = _tc_mid(acc2, hp2, degp, b2r, W3, nc)
    acc3 = edge(hp3, srcb, dstb, zerosD)
    out = _tc_final(acc3, hp3, degp, b3r, batch2, fc1_W, fc1_br, fc2_W, fc2_br, nc)
    return out


# speedup vs baseline: 21.2252x; 1.0517x over previous
import functools

import jax
import jax.numpy as jnp
from jax import lax
from jax.experimental import pallas as pl
from jax.experimental.pallas import tpu as pltpu
from jax.experimental.pallas import tpu_sc as plsc

_N = 10000
_E = 320000
_G = 64
_DH = 128
_CHUNK = 128
_ECH = 64
_WIN = 16
_NBUF = 4
_ZPAD = 128


def _mesh():
    return plsc.VectorSubcoreMesh(core_axis_name="c", subcore_axis_name="s")


def _row_split(ns):
    base = (_N // 8 // ns) * 8
    tail = _N - base * ns
    return base, tail


def _zero_acc(zeros_hbm, acc_sh, s, ns):
    base, tail = _row_split(ns)
    pltpu.sync_copy(zeros_hbm.at[pl.ds(0, base)], acc_sh.at[pl.ds(s * base, base)])
    if tail:
        @pl.when(s == ns - 1)
        def _():
            pltpu.sync_copy(
                zeros_hbm.at[pl.ds(0, tail)], acc_sh.at[pl.ds(ns * base, tail)]
            )


def _copy_out(acc_sh, out_hbm, c, s, ns):
    base, tail = _row_split(ns)
    pltpu.sync_copy(
        acc_sh.at[pl.ds(s * base, base)], out_hbm.at[c, pl.ds(s * base, base)]
    )
    if tail:
        @pl.when(s == ns - 1)
        def _():
            pltpu.sync_copy(
                acc_sh.at[pl.ds(ns * base, tail)],
                out_hbm.at[c, pl.ds(ns * base, tail)],
            )


def _sc_degree(nc, ns, ki):
    @functools.partial(
        pl.kernel,
        out_type=jax.ShapeDtypeStruct((nc, _N, _DH), jnp.float32),
        mesh=_mesh(),
        scratch_types=[
            pltpu.VMEM((ki, _CHUNK), jnp.int32),
            pltpu.VMEM((_CHUNK, _DH), jnp.float32),
            pltpu.VMEM_SHARED((_N + _CHUNK, _DH), jnp.float32),
            pltpu.SemaphoreType.DMA,
        ],
    )
    def deg_kernel(dstb_hbm, ones_hbm, zeros_hbm, out_hbm, idx_v, ones_v,
                   acc_sh, sem):
        c = lax.axis_index("c")
        s = lax.axis_index("s")
        wid = s * nc + c
        _zero_acc(zeros_hbm, acc_sh, s, ns)
        pltpu.sync_copy(dstb_hbm.at[wid], idx_v)
        pltpu.sync_copy(ones_hbm, ones_v)
        plsc.subcore_barrier()

        batch = 8
        assert ki % batch == 0

        def body(w, carry):
            ds = [
                pltpu.async_copy(
                    ones_v, acc_sh.at[idx_v.at[w * batch + u]], sem, add=True
                )
                for u in range(batch)
            ]
            for d in ds:
                d.wait()
            return carry

        lax.fori_loop(0, ki // batch, body, 0)
        plsc.subcore_barrier()
        _copy_out(acc_sh, out_hbm, c, s, ns)

    return deg_kernel


def _sc_edge(nc, ns, eki):
    assert eki % _WIN == 0 and _WIN > _NBUF

    @functools.partial(
        pl.kernel,
        out_type=jax.ShapeDtypeStruct((nc, _N, _DH), jnp.float32),
        mesh=_mesh(),
        scratch_types=[
            pltpu.VMEM((_WIN, _ECH), jnp.int32),
            pltpu.VMEM((_WIN, _ECH), jnp.int32),
            pltpu.VMEM((_ECH, _DH), jnp.float32),
            pltpu.VMEM((_ECH, _DH), jnp.float32),
            pltpu.VMEM((_ECH, _DH), jnp.float32),
            pltpu.VMEM((_ECH, _DH), jnp.float32),
            pltpu.VMEM_SHARED((_N, _DH), jnp.float32),
            pltpu.SemaphoreType.DMA,
            pltpu.SemaphoreType.DMA,
            pltpu.SemaphoreType.DMA,
            pltpu.SemaphoreType.DMA,
            pltpu.SemaphoreType.DMA,
            pltpu.SemaphoreType.DMA,
            pltpu.SemaphoreType.DMA,
            pltpu.SemaphoreType.DMA,
        ],
    )
    def edge_kernel(hp_hbm, srcb_hbm, dstb_hbm, zeros_hbm, out_hbm,
                    sidx_v, didx_v, r0, r1, r2, r3, acc_sh,
                    gs0, gs1, gs2, gs3, ss0, ss1, ss2, ss3):
        bufs = [r0, r1, r2, r3]
        gsems = [gs0, gs1, gs2, gs3]
        ssems = [ss0, ss1, ss2, ss3]
        c = lax.axis_index("c")
        s = lax.axis_index("s")
        wid = s * nc + c
        _zero_acc(zeros_hbm, acc_sh, s, ns)
        plsc.subcore_barrier()

        def window(w, carry):
            pltpu.sync_copy(srcb_hbm.at[wid, pl.ds(w * _WIN, _WIN)], sidx_v)
            pltpu.sync_copy(dstb_hbm.at[wid, pl.ds(w * _WIN, _WIN)], didx_v)

            def fire_scatter(j):
                return pltpu.async_copy(
                    bufs[j % _NBUF], acc_sh.at[didx_v.at[j]],
                    ssems[j % _NBUF], add=True,
                )

            g = [None] * _WIN
            sc = [None] * _WIN
            for j in range(_WIN):
                if j >= _NBUF:
                    sc[j - _NBUF].wait()
                g[j] = pltpu.async_copy(
                    hp_hbm.at[sidx_v.at[j]], bufs[j % _NBUF], gsems[j % _NBUF]
                )
                if j >= 1:
                    g[j - 1].wait()
                    sc[j - 1] = fire_scatter(j - 1)
            g[_WIN - 1].wait()
            sc[_WIN - 1] = fire_scatter(_WIN - 1)
            for j in range(_WIN - _NBUF, _WIN):
                sc[j].wait()
            return carry

        lax.fori_loop(0, eki // _WIN, window, 0)
        plsc.subcore_barrier()
        _copy_out(acc_sh, out_hbm, c, s, ns)

    return edge_kernel


def _dinv(degp_ref, nc):
    deg = degp_ref[0, :, 0:1]
    for i in range(1, nc):
        deg = deg + degp_ref[i, :, 0:1]
    return lax.rsqrt(deg + 1.0)


def _tc_first(x, W1, degp, nc):
    def body(x_ref, w_ref, degp_ref, out_ref):
        dinv = _dinv(degp_ref, nc)
        h = jnp.dot(x_ref[...], w_ref[...], preferred_element_type=jnp.float32)
        out_ref[pl.ds(0, _N), :] = h * dinv
        out_ref[pl.ds(_N, _ZPAD), :] = jnp.zeros((_ZPAD, _DH), jnp.float32)

    return pl.pallas_call(
        body, out_shape=jax.ShapeDtypeStruct((_N + _ZPAD, _DH), jnp.float32)
    )(x, W1, degp)


def _tc_mid(accp, hp, degp, b, W, nc):
    def body(acc_ref, hp_ref, degp_ref, b_ref, w_ref, out_ref):
        dinv = _dinv(degp_ref, nc)
        pre = hp_ref[pl.ds(0, _N), :]
        for i in range(nc):
            pre = pre + acc_ref[i]
        a = jnp.maximum(dinv * pre + b_ref[...], 0.0)
        h = jnp.dot(a, w_ref[...], preferred_element_type=jnp.float32)
        out_ref[pl.ds(0, _N), :] = h * dinv
        out_ref[pl.ds(_N, _ZPAD), :] = jnp.zeros((_ZPAD, _DH), jnp.float32)

    return pl.pallas_call(
        body, out_shape=jax.ShapeDtypeStruct((_N + _ZPAD, _DH), jnp.float32)
    )(accp, hp, degp, b, W)


def _tc_final(accp, hp, degp, b3, batch2, fc1_W, fc1_b, fc2_W, fc2_b, nc):
    def body(acc_ref, hp_ref, degp_ref, b_ref, batch_ref, w1_ref, b1_ref,
             w2_ref, b2_ref, out_ref):
        dinv = _dinv(degp_ref, nc)
        pre = hp_ref[pl.ds(0, _N), :]
        for i in range(nc):
            pre = pre + acc_ref[i]
        h = jnp.maximum(dinv * pre + b_ref[...], 0.0)
        seg = jax.lax.broadcasted_iota(jnp.int32, (_G, _N), 0) == batch_ref[...]
        pooled = jnp.dot(
            seg.astype(jnp.float32), h, preferred_element_type=jnp.float32
        )
        g = jnp.maximum(
            jnp.dot(pooled, w1_ref[...], preferred_element_type=jnp.float32)
            + b1_ref[...],
            0.0,
        )
        out_ref[...] = (
            jnp.dot(g, w2_ref[...], preferred_element_type=jnp.float32)
            + b2_ref[...]
        )

    return pl.pallas_call(
        body, out_shape=jax.ShapeDtypeStruct((_G, fc2_W.shape[1]), jnp.float32)
    )(accp, hp, degp, b3, batch2, fc1_W, fc1_b, fc2_W, fc2_b)


def kernel(x, edge_index, batch, W1, b1, W2, b2, W3, b3, fc1_W, fc1_b, fc2_W, fc2_b):
    info = plsc.get_sparse_core_info()
    nc, ns = info.num_cores, info.num_subcores
    nw = nc * ns
    src = edge_index[0].astype(jnp.int32)
    dst = edge_index[1].astype(jnp.int32)

    ki = -(-_E // (nw * _CHUNK))
    ki = -(-ki // 8) * 8
    pad = nw * ki * _CHUNK - _E
    spread = jnp.arange(pad, dtype=jnp.int32)
    dstb = jnp.concatenate([dst, _N + spread % _CHUNK]).reshape(nw, ki, _CHUNK)

    eki = -(-_E // (nw * _ECH))
    eki = -(-eki // _WIN) * _WIN
    epad = nw * eki * _ECH - _E
    espread = jnp.arange(epad, dtype=jnp.int32)
    esrcb = jnp.concatenate([src, _N + espread % _ZPAD]).reshape(nw, eki, _ECH)
    edstb = jnp.concatenate([dst, espread * 37 % _N]).reshape(nw, eki, _ECH)

    base, _ = _row_split(ns)
    onesD = jnp.ones((_CHUNK, _DH), jnp.float32)
    zerosD = jnp.zeros((base, _DH), jnp.float32)

    degp = _sc_degree(nc, ns, ki)(dstb, onesD, zerosD)

    b1r, b2r, b3r = (b.reshape(1, -1) for b in (b1, b2, b3))
    fc1_br = fc1_b.reshape(1, -1)
    fc2_br = fc2_b.reshape(1, -1)
    batch2 = batch.astype(jnp.int32).reshape(1, _N)

    edge = _sc_edge(nc, ns, eki)

    hp1 = _tc_first(x, W1, degp, nc)
    acc1 = edge(hp1, esrcb, edstb, zerosD)
    hp2 = _tc_mid(acc1, hp1, degp, b1r, W2, nc)
    acc2 = edge(hp2, esrcb, edstb, zerosD)
    hp3 = _tc_mid(acc2, hp2, degp, b2r, W3, nc)
    acc3 = edge(hp3, esrcb, edstb, zerosD)
    out = _tc_final(acc3, hp3, degp, b3r, batch2, fc1_W, fc1_br, fc2_W, fc2_br, nc)
    return out


# speedup vs baseline: 23.8516x; 1.1237x over previous
import functools

import jax
import jax.numpy as jnp
from jax import lax
from jax.experimental import pallas as pl
from jax.experimental.pallas import tpu as pltpu
from jax.experimental.pallas import tpu_sc as plsc

_N = 10000
_E = 320000
_G = 64
_DH = 128
_CHUNK = 128
_ECH = 64
_WIN = 16
_NBUF = 4
_ZPAD = 128


def _mesh():
    return plsc.VectorSubcoreMesh(core_axis_name="c", subcore_axis_name="s")


def _row_split(ns):
    base = (_N // 8 // ns) * 8
    tail = _N - base * ns
    return base, tail


def _zero_acc(zeros_hbm, acc_sh, s, ns):
    base, tail = _row_split(ns)
    pltpu.sync_copy(zeros_hbm.at[pl.ds(0, base)], acc_sh.at[pl.ds(s * base, base)])
    if tail:
        @pl.when(s == ns - 1)
        def _():
            pltpu.sync_copy(
                zeros_hbm.at[pl.ds(0, tail)], acc_sh.at[pl.ds(ns * base, tail)]
            )


def _copy_out(acc_sh, out_hbm, c, s, ns, lanes=None):
    base, tail = _row_split(ns)
    if lanes is None:
        def sl(ref, lo, n):
            return ref.at[pl.ds(lo, n)]
    else:
        def sl(ref, lo, n):
            return ref.at[pl.ds(lo, n), pl.ds(0, lanes)]

    pltpu.sync_copy(
        sl(acc_sh, s * base, base), out_hbm.at[c, pl.ds(s * base, base)]
    )
    if tail:
        @pl.when(s == ns - 1)
        def _():
            pltpu.sync_copy(
                sl(acc_sh, ns * base, tail),
                out_hbm.at[c, pl.ds(ns * base, tail)],
            )


def _sc_degree(nc, ns, ki):
    @functools.partial(
        pl.kernel,
        out_type=jax.ShapeDtypeStruct((nc, _N, _DH), jnp.float32),
        mesh=_mesh(),
        scratch_types=[
            pltpu.VMEM((ki, _CHUNK), jnp.int32),
            pltpu.VMEM((_CHUNK, _DH), jnp.float32),
            pltpu.VMEM_SHARED((_N + _CHUNK, _DH), jnp.float32),
            pltpu.SemaphoreType.DMA,
        ],
    )
    def deg_kernel(dstb_hbm, ones_hbm, zeros_hbm, out_hbm, idx_v, ones_v,
                   acc_sh, sem):
        c = lax.axis_index("c")
        s = lax.axis_index("s")
        wid = s * nc + c
        _zero_acc(zeros_hbm, acc_sh, s, ns)
        pltpu.sync_copy(dstb_hbm.at[wid], idx_v)
        pltpu.sync_copy(ones_hbm, ones_v)
        plsc.subcore_barrier()

        batch = 8
        assert ki % batch == 0

        def body(w, carry):
            ds = [
                pltpu.async_copy(
                    ones_v, acc_sh.at[idx_v.at[w * batch + u]], sem, add=True
                )
                for u in range(batch)
            ]
            for d in ds:
                d.wait()
            return carry

        lax.fori_loop(0, ki // batch, body, 0)
        plsc.subcore_barrier()
        _copy_out(acc_sh, out_hbm, c, s, ns)

    return deg_kernel


def _sc_degree_hist(nc, ns, ki):
    npad = 10240
    assert npad >= _N + _CHUNK and npad % ns == 0
    hseg = npad // ns

    @functools.partial(
        pl.kernel,
        out_type=jax.ShapeDtypeStruct((nc, _N, _DH), jnp.float32),
        mesh=_mesh(),
        compiler_params=pltpu.CompilerParams(needs_layout_passes=False),
        scratch_types=[
            pltpu.VMEM((ki, _CHUNK), jnp.int32),
            pltpu.VMEM((npad,), jnp.float32),
            pltpu.VMEM((ns, hseg), jnp.float32),
            pltpu.VMEM((hseg, _DH), jnp.float32),
            pltpu.VMEM_SHARED((ns, npad), jnp.float32),
        ],
    )
    def deg_kernel(dstb_hbm, out_hbm, idx_v, hist_v, red_v, stage_v, hists_sh):
        c = lax.axis_index("c")
        s = lax.axis_index("s")
        wid = s * nc + c
        pltpu.sync_copy(dstb_hbm.at[wid], idx_v)

        zeros16 = jnp.zeros((16,), jnp.float32)
        ones16 = jnp.ones((16,), jnp.float32)

        def zbody(i, carry):
            for u in range(8):
                hist_v[pl.ds((i * 8 + u) * 16, 16)] = zeros16
            return carry

        lax.fori_loop(0, npad // 128, zbody, 0)

        def hbody(j, carry):
            for u in range(_CHUNK // 16):
                iv = idx_v[j, pl.ds(u * 16, 16)]
                plsc.addupdate_scatter(hist_v, [iv], ones16)
            return carry

        lax.fori_loop(0, ki, hbody, 0)

        pltpu.sync_copy(hist_v, hists_sh.at[s])
        plsc.subcore_barrier()
        pltpu.sync_copy(hists_sh.at[pl.ds(0, ns), pl.ds(s * hseg, hseg)], red_v)

        lanes16 = lax.iota(jnp.int32, 16)
        zeros16i = jnp.zeros((16,), jnp.int32)

        def rbody(k, carry):
            tot = red_v[0, pl.ds(k * 16, 16)]
            for t in range(1, ns):
                tot = tot + red_v[t, pl.ds(k * 16, 16)]
            plsc.store_scatter(
                stage_v, [lanes16 + k * 16, zeros16i], tot
            )
            return carry

        lax.fori_loop(0, hseg // 16, rbody, 0)

        rows_full = hseg
        lo = s * hseg
        if _N % hseg:
            @pl.when(lo + hseg <= _N)
            def _():
                pltpu.sync_copy(
                    stage_v, out_hbm.at[c, pl.ds(lo, rows_full)]
                )

            last = _N % hseg

            @pl.when(lo + hseg > _N)
            def _():
                pltpu.sync_copy(
                    stage_v.at[pl.ds(0, last)],
                    out_hbm.at[c, pl.ds(_N - last, last)],
                )
        else:
            pltpu.sync_copy(stage_v, out_hbm.at[c, pl.ds(lo, rows_full)])

    return deg_kernel


def _sc_edge(nc, ns, eki):
    assert eki % _WIN == 0 and _WIN > _NBUF

    @functools.partial(
        pl.kernel,
        out_type=jax.ShapeDtypeStruct((nc, _N, _DH), jnp.float32),
        mesh=_mesh(),
        scratch_types=[
            pltpu.VMEM((_WIN, _ECH), jnp.int32),
            pltpu.VMEM((_WIN, _ECH), jnp.int32),
            pltpu.VMEM((_ECH, _DH), jnp.float32),
            pltpu.VMEM((_ECH, _DH), jnp.float32),
            pltpu.VMEM((_ECH, _DH), jnp.float32),
            pltpu.VMEM((_ECH, _DH), jnp.float32),
            pltpu.VMEM_SHARED((_N, _DH), jnp.float32),
            pltpu.SemaphoreType.DMA,
            pltpu.SemaphoreType.DMA,
            pltpu.SemaphoreType.DMA,
            pltpu.SemaphoreType.DMA,
            pltpu.SemaphoreType.DMA,
            pltpu.SemaphoreType.DMA,
            pltpu.SemaphoreType.DMA,
            pltpu.SemaphoreType.DMA,
        ],
    )
    def edge_kernel(hp_hbm, srcb_hbm, dstb_hbm, zeros_hbm, out_hbm,
                    sidx_v, didx_v, r0, r1, r2, r3, acc_sh,
                    gs0, gs1, gs2, gs3, ss0, ss1, ss2, ss3):
        bufs = [r0, r1, r2, r3]
        gsems = [gs0, gs1, gs2, gs3]
        ssems = [ss0, ss1, ss2, ss3]
        c = lax.axis_index("c")
        s = lax.axis_index("s")
        wid = s * nc + c
        _zero_acc(zeros_hbm, acc_sh, s, ns)
        plsc.subcore_barrier()

        def window(w, carry):
            pltpu.sync_copy(srcb_hbm.at[wid, pl.ds(w * _WIN, _WIN)], sidx_v)
            pltpu.sync_copy(dstb_hbm.at[wid, pl.ds(w * _WIN, _WIN)], didx_v)

            def fire_scatter(j):
                return pltpu.async_copy(
                    bufs[j % _NBUF], acc_sh.at[didx_v.at[j]],
                    ssems[j % _NBUF], add=True,
                )

            g = [None] * _WIN
            sc = [None] * _WIN
            for j in range(_WIN):
                if j >= _NBUF:
                    sc[j - _NBUF].wait()
                g[j] = pltpu.async_copy(
                    hp_hbm.at[sidx_v.at[j]], bufs[j % _NBUF], gsems[j % _NBUF]
                )
                if j >= 1:
                    g[j - 1].wait()
                    sc[j - 1] = fire_scatter(j - 1)
            g[_WIN - 1].wait()
            sc[_WIN - 1] = fire_scatter(_WIN - 1)
            for j in range(_WIN - _NBUF, _WIN):
                sc[j].wait()
            return carry

        lax.fori_loop(0, eki // _WIN, window, 0)
        plsc.subcore_barrier()
        _copy_out(acc_sh, out_hbm, c, s, ns)

    return edge_kernel


def _dinv(degp_ref, nc):
    deg = degp_ref[0, :, 0:1]
    for i in range(1, nc):
        deg = deg + degp_ref[i, :, 0:1]
    return lax.rsqrt(deg + 1.0)


def _tc_first(x, W1, degp, nc):
    def body(x_ref, w_ref, degp_ref, out_ref):
        dinv = _dinv(degp_ref, nc)
        h = jnp.dot(x_ref[...], w_ref[...], preferred_element_type=jnp.float32)
        out_ref[pl.ds(0, _N), :] = h * dinv
        out_ref[pl.ds(_N, _ZPAD), :] = jnp.zeros((_ZPAD, _DH), jnp.float32)

    return pl.pallas_call(
        body, out_shape=jax.ShapeDtypeStruct((_N + _ZPAD, _DH), jnp.float32)
    )(x, W1, degp)


def _tc_mid(accp, hp, degp, b, W, nc):
    def body(acc_ref, hp_ref, degp_ref, b_ref, w_ref, out_ref):
        dinv = _dinv(degp_ref, nc)
        pre = hp_ref[pl.ds(0, _N), :]
        for i in range(nc):
            pre = pre + acc_ref[i]
        a = jnp.maximum(dinv * pre + b_ref[...], 0.0)
        h = jnp.dot(a, w_ref[...], preferred_element_type=jnp.float32)
        out_ref[pl.ds(0, _N), :] = h * dinv
        out_ref[pl.ds(_N, _ZPAD), :] = jnp.zeros((_ZPAD, _DH), jnp.float32)

    return pl.pallas_call(
        body, out_shape=jax.ShapeDtypeStruct((_N + _ZPAD, _DH), jnp.float32)
    )(accp, hp, degp, b, W)


def _tc_final(accp, hp, degp, b3, batch2, fc1_W, fc1_b, fc2_W, fc2_b, nc):
    def body(acc_ref, hp_ref, degp_ref, b_ref, batch_ref, w1_ref, b1_ref,
             w2_ref, b2_ref, out_ref):
        dinv = _dinv(degp_ref, nc)
        pre = hp_ref[pl.ds(0, _N), :]
        for i in range(nc):
            pre = pre + acc_ref[i]
        h = jnp.maximum(dinv * pre + b_ref[...], 0.0)
        seg = jax.lax.broadcasted_iota(jnp.int32, (_G, _N), 0) == batch_ref[...]
        pooled = jnp.dot(
            seg.astype(jnp.float32), h, preferred_element_type=jnp.float32
        )
        g = jnp.maximum(
            jnp.dot(pooled, w1_ref[...], preferred_element_type=jnp.float32)
            + b1_ref[...],
            0.0,
        )
        out_ref[...] = (
            jnp.dot(g, w2_ref[...], preferred_element_type=jnp.float32)
            + b2_ref[...]
        )

    return pl.pallas_call(
        body, out_shape=jax.ShapeDtypeStruct((_G, fc2_W.shape[1]), jnp.float32)
    )(accp, hp, degp, b3, batch2, fc1_W, fc1_b, fc2_W, fc2_b)


def kernel(x, edge_index, batch, W1, b1, W2, b2, W3, b3, fc1_W, fc1_b, fc2_W, fc2_b):
    info = plsc.get_sparse_core_info()
    nc, ns = info.num_cores, info.num_subcores
    nw = nc * ns
    src = edge_index[0].astype(jnp.int32)
    dst = edge_index[1].astype(jnp.int32)

    ki = -(-_E // (nw * _CHUNK))
    ki = -(-ki // 8) * 8
    pad = nw * ki * _CHUNK - _E
    spread = jnp.arange(pad, dtype=jnp.int32)
    dstb = jnp.concatenate([dst, _N + spread % _CHUNK]).reshape(nw, ki, _CHUNK)

    eki = -(-_E // (nw * _ECH))
    eki = -(-eki // _WIN) * _WIN
    epad = nw * eki * _ECH - _E
    espread = jnp.arange(epad, dtype=jnp.int32)
    esrcb = jnp.concatenate([src, _N + espread % _ZPAD]).reshape(nw, eki, _ECH)
    edstb = jnp.concatenate([dst, espread * 37 % _N]).reshape(nw, eki, _ECH)

    base, _ = _row_split(ns)
    zerosD = jnp.zeros((base, _DH), jnp.float32)

    degp = _sc_degree_hist(nc, ns, ki)(dstb)

    b1r, b2r, b3r = (b.reshape(1, -1) for b in (b1, b2, b3))
    fc1_br = fc1_b.reshape(1, -1)
    fc2_br = fc2_b.reshape(1, -1)
    batch2 = batch.astype(jnp.int32).reshape(1, _N)

    edge = _sc_edge(nc, ns, eki)

    hp1 = _tc_first(x, W1, degp, nc)
    acc1 = edge(hp1, esrcb, edstb, zerosD)
    hp2 = _tc_mid(acc1, hp1, degp, b1r, W2, nc)
    acc2 = edge(hp2, esrcb, edstb, zerosD)
    hp3 = _tc_mid(acc2, hp2, degp, b2r, W3, nc)
    acc3 = edge(hp3, esrcb, edstb, zerosD)
    out = _tc_final(acc3, hp3, degp, b3r, batch2, fc1_W, fc1_br, fc2_W, fc2_br, nc)
    return out


# speedup vs baseline: 23.9112x; 1.0025x over previous
import functools

import jax
import jax.numpy as jnp
from jax import lax
from jax.experimental import pallas as pl
from jax.experimental.pallas import tpu as pltpu
from jax.experimental.pallas import tpu_sc as plsc

_N = 10000
_E = 320000
_G = 64
_DH = 128
_CHUNK = 128
_ECH = 64
_WIN = 16
_NBUF = 2
_ZPAD = 128


def _mesh():
    return plsc.VectorSubcoreMesh(core_axis_name="c", subcore_axis_name="s")


def _row_split(ns):
    base = (_N // 8 // ns) * 8
    tail = _N - base * ns
    return base, tail


def _zero_acc(zeros_hbm, acc_sh, s, ns):
    base, tail = _row_split(ns)
    pltpu.sync_copy(zeros_hbm.at[pl.ds(0, base)], acc_sh.at[pl.ds(s * base, base)])
    if tail:
        @pl.when(s == ns - 1)
        def _():
            pltpu.sync_copy(
                zeros_hbm.at[pl.ds(0, tail)], acc_sh.at[pl.ds(ns * base, tail)]
            )


def _copy_out(acc_sh, out_hbm, c, s, ns, lanes=None):
    base, tail = _row_split(ns)
    if lanes is None:
        def sl(ref, lo, n):
            return ref.at[pl.ds(lo, n)]
    else:
        def sl(ref, lo, n):
            return ref.at[pl.ds(lo, n), pl.ds(0, lanes)]

    pltpu.sync_copy(
        sl(acc_sh, s * base, base), out_hbm.at[c, pl.ds(s * base, base)]
    )
    if tail:
        @pl.when(s == ns - 1)
        def _():
            pltpu.sync_copy(
                sl(acc_sh, ns * base, tail),
                out_hbm.at[c, pl.ds(ns * base, tail)],
            )


def _sc_degree(nc, ns, ki):
    @functools.partial(
        pl.kernel,
        out_type=jax.ShapeDtypeStruct((nc, _N, _DH), jnp.float32),
        mesh=_mesh(),
        scratch_types=[
            pltpu.VMEM((ki, _CHUNK), jnp.int32),
            pltpu.VMEM((_CHUNK, _DH), jnp.float32),
            pltpu.VMEM_SHARED((_N + _CHUNK, _DH), jnp.float32),
            pltpu.SemaphoreType.DMA,
        ],
    )
    def deg_kernel(dstb_hbm, ones_hbm, zeros_hbm, out_hbm, idx_v, ones_v,
                   acc_sh, sem):
        c = lax.axis_index("c")
        s = lax.axis_index("s")
        wid = s * nc + c
        _zero_acc(zeros_hbm, acc_sh, s, ns)
        pltpu.sync_copy(dstb_hbm.at[wid], idx_v)
        pltpu.sync_copy(ones_hbm, ones_v)
        plsc.subcore_barrier()

        batch = 8
        assert ki % batch == 0

        def body(w, carry):
            ds = [
                pltpu.async_copy(
                    ones_v, acc_sh.at[idx_v.at[w * batch + u]], sem, add=True
                )
                for u in range(batch)
            ]
            for d in ds:
                d.wait()
            return carry

        lax.fori_loop(0, ki // batch, body, 0)
        plsc.subcore_barrier()
        _copy_out(acc_sh, out_hbm, c, s, ns)

    return deg_kernel


def _sc_degree_hist(nc, ns, ki):
    npad = 10240
    assert npad >= _N + _CHUNK and npad % ns == 0
    hseg = npad // ns

    @functools.partial(
        pl.kernel,
        out_type=jax.ShapeDtypeStruct((nc, _N, _DH), jnp.float32),
        mesh=_mesh(),
        compiler_params=pltpu.CompilerParams(needs_layout_passes=False),
        scratch_types=[
            pltpu.VMEM((ki, _CHUNK), jnp.int32),
            pltpu.VMEM((npad,), jnp.float32),
            pltpu.VMEM((ns, hseg), jnp.float32),
            pltpu.VMEM((hseg, _DH), jnp.float32),
            pltpu.VMEM_SHARED((ns, npad), jnp.float32),
        ],
    )
    def deg_kernel(dstb_hbm, out_hbm, idx_v, hist_v, red_v, stage_v, hists_sh):
        c = lax.axis_index("c")
        s = lax.axis_index("s")
        wid = s * nc + c
        pltpu.sync_copy(dstb_hbm.at[wid], idx_v)

        zeros16 = jnp.zeros((16,), jnp.float32)
        ones16 = jnp.ones((16,), jnp.float32)

        def zbody(i, carry):
            for u in range(8):
                hist_v[pl.ds((i * 8 + u) * 16, 16)] = zeros16
            return carry

        lax.fori_loop(0, npad // 128, zbody, 0)

        def hbody(j, carry):
            for u in range(_CHUNK // 16):
                iv = idx_v[j, pl.ds(u * 16, 16)]
                plsc.addupdate_scatter(hist_v, [iv], ones16)
            return carry

        lax.fori_loop(0, ki, hbody, 0)

        pltpu.sync_copy(hist_v, hists_sh.at[s])
        plsc.subcore_barrier()
        pltpu.sync_copy(hists_sh.at[pl.ds(0, ns), pl.ds(s * hseg, hseg)], red_v)

        lanes16 = lax.iota(jnp.int32, 16)
        zeros16i = jnp.zeros((16,), jnp.int32)

        def rbody(k, carry):
            tot = red_v[0, pl.ds(k * 16, 16)]
            for t in range(1, ns):
                tot = tot + red_v[t, pl.ds(k * 16, 16)]
            plsc.store_scatter(
                stage_v, [lanes16 + k * 16, zeros16i], tot
            )
            return carry

        lax.fori_loop(0, hseg // 16, rbody, 0)

        rows_full = hseg
        lo = s * hseg
        if _N % hseg:
            @pl.when(lo + hseg <= _N)
            def _():
                pltpu.sync_copy(
                    stage_v, out_hbm.at[c, pl.ds(lo, rows_full)]
                )

            last = _N % hseg

            @pl.when(lo + hseg > _N)
            def _():
                pltpu.sync_copy(
                    stage_v.at[pl.ds(0, last)],
                    out_hbm.at[c, pl.ds(_N - last, last)],
                )
        else:
            pltpu.sync_copy(stage_v, out_hbm.at[c, pl.ds(lo, rows_full)])

    return deg_kernel


def _sc_edge(nc, ns, eki):
    assert eki % (2 * _WIN) == 0 and _WIN > _NBUF
    hki = eki // 2

    @functools.partial(
        pl.kernel,
        out_type=jax.ShapeDtypeStruct((nc, _N, _DH), jnp.float32),
        mesh=_mesh(),
        scratch_types=[
            pltpu.VMEM((hki, _ECH), jnp.int32),
            pltpu.VMEM((hki, _ECH), jnp.int32),
            pltpu.VMEM((_ECH, _DH), jnp.float32),
            pltpu.VMEM((_ECH, _DH), jnp.float32),
            pltpu.VMEM_SHARED((_N, _DH), jnp.float32),
            pltpu.SemaphoreType.DMA,
            pltpu.SemaphoreType.DMA,
            pltpu.SemaphoreType.DMA,
            pltpu.SemaphoreType.DMA,
        ],
    )
    def edge_kernel(hp_hbm, srcb_hbm, dstb_hbm, zeros_hbm, out_hbm,
                    sidx_v, didx_v, r0, r1, acc_sh,
                    gs0, gs1, ss0, ss1):
        bufs = [r0, r1]
        gsems = [gs0, gs1]
        ssems = [ss0, ss1]
        c = lax.axis_index("c")
        s = lax.axis_index("s")
        wid = s * nc + c
        _zero_acc(zeros_hbm, acc_sh, s, ns)
        plsc.subcore_barrier()

        for half in range(2):
            pltpu.sync_copy(srcb_hbm.at[wid, pl.ds(half * hki, hki)], sidx_v)
            pltpu.sync_copy(dstb_hbm.at[wid, pl.ds(half * hki, hki)], didx_v)

            def window(w, carry):
                base = w * _WIN

                def fire_scatter(j):
                    return pltpu.async_copy(
                        bufs[j % _NBUF], acc_sh.at[didx_v.at[base + j]],
                        ssems[j % _NBUF], add=True,
                    )

                g = [None] * _WIN
                sc = [None] * _WIN
                for j in range(_WIN):
                    if j >= _NBUF:
                        sc[j - _NBUF].wait()
                    g[j] = pltpu.async_copy(
                        hp_hbm.at[sidx_v.at[base + j]], bufs[j % _NBUF],
                        gsems[j % _NBUF],
                    )
                    if j >= 1:
                        g[j - 1].wait()
                        sc[j - 1] = fire_scatter(j - 1)
                g[_WIN - 1].wait()
                sc[_WIN - 1] = fire_scatter(_WIN - 1)
                for j in range(_WIN - _NBUF, _WIN):
                    sc[j].wait()
                return carry

            lax.fori_loop(0, hki // _WIN, window, 0)

        plsc.subcore_barrier()
        _copy_out(acc_sh, out_hbm, c, s, ns)

    return edge_kernel


def _dinv(degp_ref, nc):
    deg = degp_ref[0, :, 0:1]
    for i in range(1, nc):
        deg = deg + degp_ref[i, :, 0:1]
    return lax.rsqrt(deg + 1.0)


def _tc_first(x, W1, degp, nc):
    def body(x_ref, w_ref, degp_ref, out_ref, dinv_ref):
        dinv = _dinv(degp_ref, nc)
        h = jnp.dot(x_ref[...], w_ref[...], preferred_element_type=jnp.float32)
        out_ref[pl.ds(0, _N), :] = h * dinv
        out_ref[pl.ds(_N, _ZPAD), :] = jnp.zeros((_ZPAD, _DH), jnp.float32)
        dinv_ref[...] = dinv

    return pl.pallas_call(
        body,
        out_shape=(
            jax.ShapeDtypeStruct((_N + _ZPAD, _DH), jnp.float32),
            jax.ShapeDtypeStruct((_N, 1), jnp.float32),
        ),
    )(x, W1, degp)


def _tc_mid(accp, hp, dinv_col, b, W, nc):
    def body(acc_ref, hp_ref, dinv_ref, b_ref, w_ref, out_ref):
        dinv = dinv_ref[...]
        pre = hp_ref[pl.ds(0, _N), :]
        for i in range(nc):
            pre = pre + acc_ref[i]
        a = jnp.maximum(dinv * pre + b_ref[...], 0.0)
        h = jnp.dot(a, w_ref[...], preferred_element_type=jnp.float32)
        out_ref[pl.ds(0, _N), :] = h * dinv
        out_ref[pl.ds(_N, _ZPAD), :] = jnp.zeros((_ZPAD, _DH), jnp.float32)

    return pl.pallas_call(
        body, out_shape=jax.ShapeDtypeStruct((_N + _ZPAD, _DH), jnp.float32)
    )(accp, hp, dinv_col, b, W)


def _tc_final(accp, hp, dinv_col, b3, batch2, fc1_W, fc1_b, fc2_W, fc2_b, nc):
    def body(acc_ref, hp_ref, dinv_ref, b_ref, batch_ref, w1_ref, b1_ref,
             w2_ref, b2_ref, out_ref):
        dinv = dinv_ref[...]
        pre = hp_ref[pl.ds(0, _N), :]
        for i in range(nc):
            pre = pre + acc_ref[i]
        h = jnp.maximum(dinv * pre + b_ref[...], 0.0)
        seg = jax.lax.broadcasted_iota(jnp.int32, (_G, _N), 0) == batch_ref[...]
        pooled = jnp.dot(
            seg.astype(jnp.float32), h, preferred_element_type=jnp.float32
        )
        g = jnp.maximum(
            jnp.dot(pooled, w1_ref[...], preferred_element_type=jnp.float32)
            + b1_ref[...],
            0.0,
        )
        out_ref[...] = (
            jnp.dot(g, w2_ref[...], preferred_element_type=jnp.float32)
            + b2_ref[...]
        )

    return pl.pallas_call(
        body, out_shape=jax.ShapeDtypeStruct((_G, fc2_W.shape[1]), jnp.float32)
    )(accp, hp, dinv_col, b3, batch2, fc1_W, fc1_b, fc2_W, fc2_b)


def kernel(x, edge_index, batch, W1, b1, W2, b2, W3, b3, fc1_W, fc1_b, fc2_W, fc2_b):
    info = plsc.get_sparse_core_info()
    nc, ns = info.num_cores, info.num_subcores
    nw = nc * ns
    src = edge_index[0].astype(jnp.int32)
    dst = edge_index[1].astype(jnp.int32)

    ki = -(-_E // (nw * _CHUNK))
    ki = -(-ki // 8) * 8
    pad = nw * ki * _CHUNK - _E
    spread = jnp.arange(pad, dtype=jnp.int32)
    dstb = jnp.concatenate([dst, _N + spread % _CHUNK]).reshape(nw, ki, _CHUNK)

    eki = -(-_E // (nw * _ECH))
    eki = -(-eki // _WIN) * _WIN
    epad = nw * eki * _ECH - _E
    espread = jnp.arange(epad, dtype=jnp.int32)
    esrcb = jnp.concatenate([src, _N + espread % _ZPAD]).reshape(nw, eki, _ECH)
    edstb = jnp.concatenate([dst, espread * 37 % _N]).reshape(nw, eki, _ECH)

    base, _ = _row_split(ns)
    zerosD = jnp.zeros((base, _DH), jnp.float32)

    degp = _sc_degree_hist(nc, ns, ki)(dstb)

    b1r, b2r, b3r = (b.reshape(1, -1) for b in (b1, b2, b3))
    fc1_br = fc1_b.reshape(1, -1)
    fc2_br = fc2_b.reshape(1, -1)
    batch2 = batch.astype(jnp.int32).reshape(1, _N)

    edge = _sc_edge(nc, ns, eki)

    hp1, dinv_col = _tc_first(x, W1, degp, nc)
    acc1 = edge(hp1, esrcb, edstb, zerosD)
    hp2 = _tc_mid(acc1, hp1, dinv_col, b1r, W2, nc)
    acc2 = edge(hp2, esrcb, edstb, zerosD)
    hp3 = _tc_mid(acc2, hp2, dinv_col, b2r, W3, nc)
    acc3 = edge(hp3, esrcb, edstb, zerosD)
    out = _tc_final(acc3, hp3, dinv_col, b3r, batch2, fc1_W, fc1_br, fc2_W,
                    fc2_br, nc)
    return out


# speedup vs baseline: 23.9249x; 1.0006x over previous
import functools

import jax
import jax.numpy as jnp
from jax import lax
from jax.experimental import pallas as pl
from jax.experimental.pallas import tpu as pltpu
from jax.experimental.pallas import tpu_sc as plsc

_N = 10000
_E = 320000
_G = 64
_DH = 128
_CHUNK = 128
_ECH = 64
_WIN = 16
_NBUF = 2
_ZPAD = 128


def _mesh():
    return plsc.VectorSubcoreMesh(core_axis_name="c", subcore_axis_name="s")


def _row_split(ns):
    base = (_N // 8 // ns) * 8
    tail = _N - base * ns
    return base, tail


def _zero_acc(zeros_hbm, acc_sh, s, ns):
    base, tail = _row_split(ns)
    pltpu.sync_copy(zeros_hbm.at[pl.ds(0, base)], acc_sh.at[pl.ds(s * base, base)])
    if tail:
        @pl.when(s == ns - 1)
        def _():
            pltpu.sync_copy(
                zeros_hbm.at[pl.ds(0, tail)], acc_sh.at[pl.ds(ns * base, tail)]
            )


def _copy_out(acc_sh, out_hbm, c, s, ns, lanes=None):
    base, tail = _row_split(ns)
    if lanes is None:
        def sl(ref, lo, n):
            return ref.at[pl.ds(lo, n)]
    else:
        def sl(ref, lo, n):
            return ref.at[pl.ds(lo, n), pl.ds(0, lanes)]

    pltpu.sync_copy(
        sl(acc_sh, s * base, base), out_hbm.at[c, pl.ds(s * base, base)]
    )
    if tail:
        @pl.when(s == ns - 1)
        def _():
            pltpu.sync_copy(
                sl(acc_sh, ns * base, tail),
                out_hbm.at[c, pl.ds(ns * base, tail)],
            )


def _sc_degree(nc, ns, ki):
    @functools.partial(
        pl.kernel,
        out_type=jax.ShapeDtypeStruct((nc, _N, _DH), jnp.float32),
        mesh=_mesh(),
        scratch_types=[
            pltpu.VMEM((ki, _CHUNK), jnp.int32),
            pltpu.VMEM((_CHUNK, _DH), jnp.float32),
            pltpu.VMEM_SHARED((_N + _CHUNK, _DH), jnp.float32),
            pltpu.SemaphoreType.DMA,
        ],
    )
    def deg_kernel(dstb_hbm, ones_hbm, zeros_hbm, out_hbm, idx_v, ones_v,
                   acc_sh, sem):
        c = lax.axis_index("c")
        s = lax.axis_index("s")
        wid = s * nc + c
        _zero_acc(zeros_hbm, acc_sh, s, ns)
        pltpu.sync_copy(dstb_hbm.at[wid], idx_v)
        pltpu.sync_copy(ones_hbm, ones_v)
        plsc.subcore_barrier()

        batch = 8
        assert ki % batch == 0

        def body(w, carry):
            ds = [
                pltpu.async_copy(
                    ones_v, acc_sh.at[idx_v.at[w * batch + u]], sem, add=True
                )
                for u in range(batch)
            ]
            for d in ds:
                d.wait()
            return carry

        lax.fori_loop(0, ki // batch, body, 0)
        plsc.subcore_barrier()
        _copy_out(acc_sh, out_hbm, c, s, ns)

    return deg_kernel


def _sc_degree_hist(nc, ns, ki):
    npad = 10240
    assert npad >= _N + _CHUNK and npad % ns == 0
    hseg = npad // ns

    @functools.partial(
        pl.kernel,
        out_type=jax.ShapeDtypeStruct((nc, _N, _DH), jnp.float32),
        mesh=_mesh(),
        compiler_params=pltpu.CompilerParams(needs_layout_passes=False, skip_device_barrier=True),
        scratch_types=[
            pltpu.VMEM((ki, _CHUNK), jnp.int32),
            pltpu.VMEM((npad,), jnp.float32),
            pltpu.VMEM((ns, hseg), jnp.float32),
            pltpu.VMEM((hseg, _DH), jnp.float32),
            pltpu.VMEM_SHARED((ns, npad), jnp.float32),
        ],
    )
    def deg_kernel(dstb_hbm, out_hbm, idx_v, hist_v, red_v, stage_v, hists_sh):
        c = lax.axis_index("c")
        s = lax.axis_index("s")
        wid = s * nc + c
        pltpu.sync_copy(dstb_hbm.at[wid], idx_v)

        zeros16 = jnp.zeros((16,), jnp.float32)
        ones16 = jnp.ones((16,), jnp.float32)

        def zbody(i, carry):
            for u in range(8):
                hist_v[pl.ds((i * 8 + u) * 16, 16)] = zeros16
            return carry

        lax.fori_loop(0, npad // 128, zbody, 0)

        def hbody(j, carry):
            for u in range(_CHUNK // 16):
                iv = idx_v[j, pl.ds(u * 16, 16)]
                plsc.addupdate_scatter(hist_v, [iv], ones16)
            return carry

        lax.fori_loop(0, ki, hbody, 0)

        pltpu.sync_copy(hist_v, hists_sh.at[s])
        plsc.subcore_barrier()
        pltpu.sync_copy(hists_sh.at[pl.ds(0, ns), pl.ds(s * hseg, hseg)], red_v)

        lanes16 = lax.iota(jnp.int32, 16)
        zeros16i = jnp.zeros((16,), jnp.int32)

        def rbody(k, carry):
            tot = red_v[0, pl.ds(k * 16, 16)]
            for t in range(1, ns):
                tot = tot + red_v[t, pl.ds(k * 16, 16)]
            plsc.store_scatter(
                stage_v, [lanes16 + k * 16, zeros16i], tot
            )
            return carry

        lax.fori_loop(0, hseg // 16, rbody, 0)

        rows_full = hseg
        lo = s * hseg
        if _N % hseg:
            @pl.when(lo + hseg <= _N)
            def _():
                pltpu.sync_copy(
                    stage_v, out_hbm.at[c, pl.ds(lo, rows_full)]
                )

            last = _N % hseg

            @pl.when(lo + hseg > _N)
            def _():
                pltpu.sync_copy(
                    stage_v.at[pl.ds(0, last)],
                    out_hbm.at[c, pl.ds(_N - last, last)],
                )
        else:
            pltpu.sync_copy(stage_v, out_hbm.at[c, pl.ds(lo, rows_full)])

    return deg_kernel


def _sc_edge(nc, ns, eki):
    assert eki % (2 * _WIN) == 0 and _WIN > _NBUF
    hki = eki // 2

    @functools.partial(
        pl.kernel,
        out_type=jax.ShapeDtypeStruct((nc, _N, _DH), jnp.float32),
        mesh=_mesh(),
        compiler_params=pltpu.CompilerParams(skip_device_barrier=True),
        scratch_types=[
            pltpu.VMEM((hki, _ECH), jnp.int32),
            pltpu.VMEM((hki, _ECH), jnp.int32),
            pltpu.VMEM((_ECH, _DH), jnp.float32),
            pltpu.VMEM((_ECH, _DH), jnp.float32),
            pltpu.VMEM_SHARED((_N, _DH), jnp.float32),
            pltpu.SemaphoreType.DMA,
            pltpu.SemaphoreType.DMA,
            pltpu.SemaphoreType.DMA,
            pltpu.SemaphoreType.DMA,
        ],
    )
    def edge_kernel(hp_hbm, srcb_hbm, dstb_hbm, zeros_hbm, out_hbm,
                    sidx_v, didx_v, r0, r1, acc_sh,
                    gs0, gs1, ss0, ss1):
        bufs = [r0, r1]
        gsems = [gs0, gs1]
        ssems = [ss0, ss1]
        c = lax.axis_index("c")
        s = lax.axis_index("s")
        wid = s * nc + c
        _zero_acc(zeros_hbm, acc_sh, s, ns)
        plsc.subcore_barrier()

        for half in range(2):
            pltpu.sync_copy(srcb_hbm.at[wid, pl.ds(half * hki, hki)], sidx_v)
            pltpu.sync_copy(dstb_hbm.at[wid, pl.ds(half * hki, hki)], didx_v)

            def window(w, carry):
                base = w * _WIN

                def fire_scatter(j):
                    return pltpu.async_copy(
                        bufs[j % _NBUF], acc_sh.at[didx_v.at[base + j]],
                        ssems[j % _NBUF], add=True,
                    )

                g = [None] * _WIN
                sc = [None] * _WIN
                for j in range(_WIN):
                    if j >= _NBUF:
                        sc[j - _NBUF].wait()
                    g[j] = pltpu.async_copy(
                        hp_hbm.at[sidx_v.at[base + j]], bufs[j % _NBUF],
                        gsems[j % _NBUF],
                    )
                    if j >= 1:
                        g[j - 1].wait()
                        sc[j - 1] = fire_scatter(j - 1)
                g[_WIN - 1].wait()
                sc[_WIN - 1] = fire_scatter(_WIN - 1)
                for j in range(_WIN - _NBUF, _WIN):
                    sc[j].wait()
                return carry

            lax.fori_loop(0, hki // _WIN, window, 0)

        plsc.subcore_barrier()
        _copy_out(acc_sh, out_hbm, c, s, ns)

    return edge_kernel


def _dinv(degp_ref, nc):
    deg = degp_ref[0, :, 0:1]
    for i in range(1, nc):
        deg = deg + degp_ref[i, :, 0:1]
    return lax.rsqrt(deg + 1.0)


def _tc_first(x, W1, degp, nc):
    def body(x_ref, w_ref, degp_ref, out_ref, dinv_ref):
        dinv = _dinv(degp_ref, nc)
        h = jnp.dot(x_ref[...], w_ref[...], preferred_element_type=jnp.float32)
        out_ref[pl.ds(0, _N), :] = h * dinv
        out_ref[pl.ds(_N, _ZPAD), :] = jnp.zeros((_ZPAD, _DH), jnp.float32)
        dinv_ref[...] = dinv

    return pl.pallas_call(
        body,
        out_shape=(
            jax.ShapeDtypeStruct((_N + _ZPAD, _DH), jnp.float32),
            jax.ShapeDtypeStruct((_N, 1), jnp.float32),
        ),
    )(x, W1, degp)


def _tc_mid(accp, hp, dinv_col, b, W, nc):
    def body(acc_ref, hp_ref, dinv_ref, b_ref, w_ref, out_ref):
        dinv = dinv_ref[...]
        pre = hp_ref[pl.ds(0, _N), :]
        for i in range(nc):
            pre = pre + acc_ref[i]
        a = jnp.maximum(dinv * pre + b_ref[...], 0.0)
        h = jnp.dot(a, w_ref[...], preferred_element_type=jnp.float32)
        out_ref[pl.ds(0, _N), :] = h * dinv
        out_ref[pl.ds(_N, _ZPAD), :] = jnp.zeros((_ZPAD, _DH), jnp.float32)

    return pl.pallas_call(
        body, out_shape=jax.ShapeDtypeStruct((_N + _ZPAD, _DH), jnp.float32)
    )(accp, hp, dinv_col, b, W)


def _tc_final(accp, hp, dinv_col, b3, batch2, fc1_W, fc1_b, fc2_W, fc2_b, nc):
    def body(acc_ref, hp_ref, dinv_ref, b_ref, batch_ref, w1_ref, b1_ref,
             w2_ref, b2_ref, out_ref):
        dinv = dinv_ref[...]
        pre = hp_ref[pl.ds(0, _N), :]
        for i in range(nc):
            pre = pre + acc_ref[i]
        h = jnp.maximum(dinv * pre + b_ref[...], 0.0)
        seg = jax.lax.broadcasted_iota(jnp.int32, (_G, _N), 0) == batch_ref[...]
        pooled = jnp.dot(
            seg.astype(jnp.float32), h, preferred_element_type=jnp.float32
        )
        g = jnp.maximum(
            jnp.dot(pooled, w1_ref[...], preferred_element_type=jnp.float32)
            + b1_ref[...],
            0.0,
        )
        out_ref[...] = (
            jnp.dot(g, w2_ref[...], preferred_element_type=jnp.float32)
            + b2_ref[...]
        )

    return pl.pallas_call(
        body, out_shape=jax.ShapeDtypeStruct((_G, fc2_W.shape[1]), jnp.float32)
    )(accp, hp, dinv_col, b3, batch2, fc1_W, fc1_b, fc2_W, fc2_b)


def kernel(x, edge_index, batch, W1, b1, W2, b2, W3, b3, fc1_W, fc1_b, fc2_W, fc2_b):
    info = plsc.get_sparse_core_info()
    nc, ns = info.num_cores, info.num_subcores
    nw = nc * ns
    src = edge_index[0].astype(jnp.int32)
    dst = edge_index[1].astype(jnp.int32)

    ki = -(-_E // (nw * _CHUNK))
    ki = -(-ki // 8) * 8
    pad = nw * ki * _CHUNK - _E
    spread = jnp.arange(pad, dtype=jnp.int32)
    dstb = jnp.concatenate([dst, _N + spread % _CHUNK]).reshape(nw, ki, _CHUNK)

    eki = -(-_E // (nw * _ECH))
    eki = -(-eki // _WIN) * _WIN
    epad = nw * eki * _ECH - _E
    espread = jnp.arange(epad, dtype=jnp.int32)
    esrcb = jnp.concatenate([src, _N + espread % _ZPAD]).reshape(nw, eki, _ECH)
    edstb = jnp.concatenate([dst, espread * 37 % _N]).reshape(nw, eki, _ECH)

    base, _ = _row_split(ns)
    zerosD = jnp.zeros((base, _DH), jnp.float32)

    degp = _sc_degree_hist(nc, ns, ki)(dstb)

    b1r, b2r, b3r = (b.reshape(1, -1) for b in (b1, b2, b3))
    fc1_br = fc1_b.reshape(1, -1)
    fc2_br = fc2_b.reshape(1, -1)
    batch2 = batch.astype(jnp.int32).reshape(1, _N)

    edge = _sc_edge(nc, ns, eki)

    hp1, dinv_col = _tc_first(x, W1, degp, nc)
    acc1 = edge(hp1, esrcb, edstb, zerosD)
    hp2 = _tc_mid(acc1, hp1, dinv_col, b1r, W2, nc)
    acc2 = edge(hp2, esrcb, edstb, zerosD)
    hp3 = _tc_mid(acc2, hp2, dinv_col, b2r, W3, nc)
    acc3 = edge(hp3, esrcb, edstb, zerosD)
    out = _tc_final(acc3, hp3, dinv_col, b3r, batch2, fc1_W, fc1_br, fc2_W,
                    fc2_br, nc)
    return out
